# baseline (speedup 1.0000x reference)
import functools
import jax
import jax.numpy as jnp
from jax import lax
from jax.experimental import pallas as pl
from jax.experimental.pallas import tpu as pltpu
from jax.experimental.pallas import tpu_sc as plsc

_K = 64
_ROWS = 64
_COLS = 32768
_NVEC = _COLS // 16
_NC = 2
_NS = 16
_ROWS_PER_TILE = _ROWS // (_NC * _NS)
_HISTW = 256 * 16


def _keyify(v):
    b = plsc.bitcast(v, jnp.int32)
    return b ^ (jnp.right_shift(b, 31) & jnp.int32(0x7FFFFFFF))


def _splat16(x):
    return lax.broadcast_in_dim(x, (16,), ())


def _lshr(x, n):
    u = plsc.bitcast(x, jnp.uint32)
    return plsc.bitcast(jnp.right_shift(u, jnp.uint32(n)), jnp.int32)


def _find_bucket(tot_v, lane, k, base=None):
    if base is None:
        base = jnp.int32(0)
    best = jnp.int32(-1)
    run = jnp.int32(0)
    for j in range(15, -1, -1):
        h = tot_v[pl.ds(base + jnp.int32(j * 16), 16)]
        suf = lax.rev(jnp.cumsum(lax.rev(h, (0,))), (0,)) + run
        run = run + jnp.sum(h)
        cand = jnp.where(suf >= k, lane + jnp.int32(j * 16), jnp.int32(-1))
        best = jnp.maximum(best, jnp.max(cand))
    B = best
    jb = jnp.right_shift(B, 4)
    lb = B & jnp.int32(15)
    tb = tot_v[pl.ds(base + jb * 16, 16)]
    neg = jnp.int32(-(2**31))
    hist_b = jnp.max(jnp.where(lane == lb, tb, neg))
    acc = jnp.zeros((16,), jnp.int32)
    for j in range(16):
        h = tot_v[pl.ds(base + jnp.int32(j * 16), 16)]
        bidx = lane + jnp.int32(j * 16)
        acc = acc + jnp.where(bidx >= B, h, jnp.int32(0))
    count_ge = jnp.sum(acc)
    return B, k - (count_ge - hist_b)


def _find_bucket12(h_v, lane, k):
    neg = jnp.int32(-(2**31))
    tots = jnp.zeros((16,), jnp.int32)
    for J in range(16):
        acc = h_v[pl.ds(J * 256, 16)]
        for v in range(1, 16):
            acc = acc + h_v[pl.ds(J * 256 + v * 16, 16)]
        tots = jnp.where(lane == J, jnp.sum(acc), tots)
    suf = lax.rev(jnp.cumsum(lax.rev(tots, (0,))), (0,))
    J = jnp.max(jnp.where(suf >= k, lane, jnp.int32(-1)))
    suf_j = jnp.max(jnp.where(lane == J, suf, neg))
    tot_j = jnp.max(jnp.where(lane == J, tots, neg))
    k2 = k - (suf_j - tot_j)
    b_rel, k3 = _find_bucket(h_v, lane, k2, base=lax.shift_left(J, jnp.int32(8)))
    return lax.shift_left(J, jnp.int32(8)) | b_rel, k3


def _reduce_hist(hist_v, tot_v):
    for j in range(16):
        acc = hist_v[pl.ds(j * 16, 16)]
        for l in range(1, 16):
            acc = acc + hist_v[pl.ds(l * 256 + j * 16, 16)]
        tot_v[pl.ds(j * 16, 16)] = acc


def _zero_hist(hist_v):
    zero16 = jnp.zeros((16,), jnp.int32)

    @plsc.parallel_loop(0, _HISTW // 16, unroll=8)
    def _(i):
        hist_v[pl.ds(i * 16, 16)] = zero16


def _process_row(row_v, hist_v, tot_v, lane, lane_base, ones16, ninf16):
    kq = jnp.int32(_K)

    _zero_hist(hist_v)
    lane_base128 = lane_base + jnp.int32(128)

    @plsc.parallel_loop(0, _NVEC, unroll=16)
    def _(i):
        key = _keyify(row_v[pl.ds(i * 16, 16)])
        digit = jnp.right_shift(key, jnp.int32(24))
        plsc.addupdate_scatter(hist_v, [lane_base128 + digit], ones16)

    _reduce_hist(hist_v, tot_v)
    B, k = _find_bucket(tot_v, lane, kq)
    prefix = B ^ jnp.int32(0x80)

    for hi_shift, dg_shift in ((24, 12), (12, 0)):
        pref = prefix
        _zero_hist(hist_v)

        @plsc.parallel_loop(0, _NVEC, unroll=16)
        def _(i):
            key = _keyify(row_v[pl.ds(i * 16, 16)])
            m = _lshr(key, hi_shift) == pref
            digit = _lshr(key, dg_shift) & jnp.int32(0xFFF) if dg_shift else key & jnp.int32(0xFFF)
            plsc.addupdate_scatter(hist_v, [digit], ones16, mask=m)

        B, k = _find_bucket12(hist_v, lane, k)
        prefix = lax.shift_left(prefix, jnp.int32(12)) | B

    tbits = prefix ^ (jnp.right_shift(prefix, 31) & jnp.int32(0x7FFFFFFF))
    tf = plsc.bitcast(_splat16(tbits), jnp.float32)

    @plsc.parallel_loop(0, _NVEC, unroll=16)
    def _(i):
        v = row_v[pl.ds(i * 16, 16)]
        row_v[pl.ds(i * 16, 16)] = jnp.where(v < tf, ninf16, v)


def _sc_body(x_hbm, o_hbm, row_a, row_b, hist_v, tot_v, sem_a, sem_b):
    wid = lax.axis_index("s") * _NC + lax.axis_index("c")
    lane = lax.iota(jnp.int32, 16)
    lane_base = lane * jnp.int32(256)
    ones16 = jnp.ones((16,), jnp.int32)
    ninf16 = jnp.full((16,), -jnp.inf, jnp.float32)

    row0 = wid * _ROWS_PER_TILE
    row1 = row0 + 1
    cp_a = pltpu.async_copy(x_hbm.at[row0], row_a, sem_a)
    cp_b = pltpu.async_copy(x_hbm.at[row1], row_b, sem_b)
    cp_a.wait()
    _process_row(row_a, hist_v, tot_v, lane, lane_base, ones16, ninf16)
    st_a = pltpu.async_copy(row_a, o_hbm.at[row0], sem_a)
    cp_b.wait()
    _process_row(row_b, hist_v, tot_v, lane, lane_base, ones16, ninf16)
    st_a.wait()
    pltpu.sync_copy(row_b, o_hbm.at[row1])


def kernel(x):
    mesh = plsc.VectorSubcoreMesh(core_axis_name="c", subcore_axis_name="s")
    f = functools.partial(
        pl.kernel,
        mesh=mesh,
        out_type=jax.ShapeDtypeStruct((_ROWS, _COLS), jnp.float32),
        compiler_params=pltpu.CompilerParams(
            needs_layout_passes=False,
            disable_bounds_checks=True,
            disable_semaphore_checks=True,
        ),
        scratch_types=[
            pltpu.VMEM((_COLS,), jnp.float32),
            pltpu.VMEM((_COLS,), jnp.float32),
            pltpu.VMEM((_HISTW,), jnp.int32),
            pltpu.VMEM((256,), jnp.int32),
            pltpu.SemaphoreType.DMA,
            pltpu.SemaphoreType.DMA,
        ],
    )(_sc_body)
    return f(x)


# speedup vs baseline: 1.0127x; 1.0127x over previous
import functools
import jax
import jax.numpy as jnp
from jax import lax
from jax.experimental import pallas as pl
from jax.experimental.pallas import tpu as pltpu
from jax.experimental.pallas import tpu_sc as plsc

_K = 64
_ROWS = 64
_COLS = 32768
_NVEC = _COLS // 16
_NC = 2
_NS = 16
_ROWS_PER_TILE = _ROWS // (_NC * _NS)
_HISTW = 256 * 16


def _keyify(v):
    b = plsc.bitcast(v, jnp.int32)
    return b ^ (jnp.right_shift(b, 31) & jnp.int32(0x7FFFFFFF))


def _splat16(x):
    return lax.broadcast_in_dim(x, (16,), ())


def _lshr(x, n):
    u = plsc.bitcast(x, jnp.uint32)
    return plsc.bitcast(jnp.right_shift(u, jnp.uint32(n)), jnp.int32)


def _find_bucket(tot_v, lane, k, base=None):
    if base is None:
        base = jnp.int32(0)
    best = jnp.int32(-1)
    run = jnp.int32(0)
    for j in range(15, -1, -1):
        h = tot_v[pl.ds(base + jnp.int32(j * 16), 16)]
        suf = lax.rev(jnp.cumsum(lax.rev(h, (0,))), (0,)) + run
        run = run + jnp.sum(h)
        cand = jnp.where(suf >= k, lane + jnp.int32(j * 16), jnp.int32(-1))
        best = jnp.maximum(best, jnp.max(cand))
    B = best
    jb = jnp.right_shift(B, 4)
    lb = B & jnp.int32(15)
    tb = tot_v[pl.ds(base + jb * 16, 16)]
    neg = jnp.int32(-(2**31))
    hist_b = jnp.max(jnp.where(lane == lb, tb, neg))
    acc = jnp.zeros((16,), jnp.int32)
    for j in range(16):
        h = tot_v[pl.ds(base + jnp.int32(j * 16), 16)]
        bidx = lane + jnp.int32(j * 16)
        acc = acc + jnp.where(bidx >= B, h, jnp.int32(0))
    count_ge = jnp.sum(acc)
    return B, k - (count_ge - hist_b)


def _find_bucket12(h_v, lane, k):
    neg = jnp.int32(-(2**31))
    tots = jnp.zeros((16,), jnp.int32)
    for J in range(16):
        acc = h_v[pl.ds(J * 256, 16)]
        for v in range(1, 16):
            acc = acc + h_v[pl.ds(J * 256 + v * 16, 16)]
        tots = jnp.where(lane == J, jnp.sum(acc), tots)
    suf = lax.rev(jnp.cumsum(lax.rev(tots, (0,))), (0,))
    J = jnp.max(jnp.where(suf >= k, lane, jnp.int32(-1)))
    suf_j = jnp.max(jnp.where(lane == J, suf, neg))
    tot_j = jnp.max(jnp.where(lane == J, tots, neg))
    k2 = k - (suf_j - tot_j)
    b_rel, k3 = _find_bucket(h_v, lane, k2, base=lax.shift_left(J, jnp.int32(8)))
    return lax.shift_left(J, jnp.int32(8)) | b_rel, k3


def _reduce_hist(hist_v, tot_v):
    for j in range(16):
        acc = hist_v[pl.ds(j * 16, 16)]
        for l in range(1, 16):
            acc = acc + hist_v[pl.ds(l * 256 + j * 16, 16)]
        tot_v[pl.ds(j * 16, 16)] = acc


def _zero_hist(hist_v):
    zero16 = jnp.zeros((16,), jnp.int32)

    @plsc.parallel_loop(0, _HISTW // 16, unroll=8)
    def _(i):
        hist_v[pl.ds(i * 16, 16)] = zero16


def _process_row(row_v, hist_v, tot_v, lane, lane_base, ones16, ninf16):
    kq = jnp.int32(_K)

    _zero_hist(hist_v)
    lane_base128 = lane_base + jnp.int32(128)

    @plsc.parallel_loop(0, _NVEC, unroll=16)
    def _(i):
        key = _keyify(row_v[pl.ds(i * 16, 16)])
        digit = jnp.right_shift(key, jnp.int32(24))
        plsc.addupdate_scatter(hist_v, [lane_base128 + digit], ones16)

    _reduce_hist(hist_v, tot_v)
    B, k = _find_bucket(tot_v, lane, kq)
    prefix = B ^ jnp.int32(0x80)

    sign_neg = prefix >= jnp.int32(128)
    sm = jnp.where(sign_neg, jnp.int32(0xFFF), jnp.int32(0))
    rawp = prefix ^ jnp.where(sign_neg, jnp.int32(0x7F), jnp.int32(0))
    for hi_shift, dg_shift in ((24, 12), (12, 0)):
        rp = rawp
        _zero_hist(hist_v)

        @plsc.parallel_loop(0, _NVEC, unroll=16)
        def _(i):
            b = plsc.bitcast(row_v[pl.ds(i * 16, 16)], jnp.int32)
            m = _lshr(b, hi_shift) == rp
            d = _lshr(b, dg_shift) & jnp.int32(0xFFF) if dg_shift else b & jnp.int32(0xFFF)
            plsc.addupdate_scatter(hist_v, [d ^ sm], ones16, mask=m)

        B, k = _find_bucket12(hist_v, lane, k)
        rawp = lax.shift_left(rawp, jnp.int32(12)) | (B ^ sm)

    tf = plsc.bitcast(_splat16(rawp), jnp.float32)

    @plsc.parallel_loop(0, _NVEC, unroll=16)
    def _(i):
        v = row_v[pl.ds(i * 16, 16)]
        row_v[pl.ds(i * 16, 16)] = jnp.where(v < tf, ninf16, v)


def _sc_body(x_hbm, o_hbm, row_a, row_b, hist_v, tot_v, sem_a, sem_b):
    wid = lax.axis_index("s") * _NC + lax.axis_index("c")
    lane = lax.iota(jnp.int32, 16)
    lane_base = lane * jnp.int32(256)
    ones16 = jnp.ones((16,), jnp.int32)
    ninf16 = jnp.full((16,), -jnp.inf, jnp.float32)

    row0 = wid * _ROWS_PER_TILE
    row1 = row0 + 1
    cp_a = pltpu.async_copy(x_hbm.at[row0], row_a, sem_a)
    cp_b = pltpu.async_copy(x_hbm.at[row1], row_b, sem_b)
    cp_a.wait()
    _process_row(row_a, hist_v, tot_v, lane, lane_base, ones16, ninf16)
    st_a = pltpu.async_copy(row_a, o_hbm.at[row0], sem_a)
    cp_b.wait()
    _process_row(row_b, hist_v, tot_v, lane, lane_base, ones16, ninf16)
    st_a.wait()
    pltpu.sync_copy(row_b, o_hbm.at[row1])


def kernel(x):
    mesh = plsc.VectorSubcoreMesh(core_axis_name="c", subcore_axis_name="s")
    f = functools.partial(
        pl.kernel,
        mesh=mesh,
        out_type=jax.ShapeDtypeStruct((_ROWS, _COLS), jnp.float32),
        compiler_params=pltpu.CompilerParams(needs_layout_passes=False),
        scratch_types=[
            pltpu.VMEM((_COLS,), jnp.float32),
            pltpu.VMEM((_COLS,), jnp.float32),
            pltpu.VMEM((_HISTW,), jnp.int32),
            pltpu.VMEM((256,), jnp.int32),
            pltpu.SemaphoreType.DMA,
            pltpu.SemaphoreType.DMA,
        ],
    )(_sc_body)
    return f(x)
